# baseline (speedup 1.0000x reference)
import functools

import jax
import jax.numpy as jnp
from jax import lax
from jax.experimental import pallas as pl
from jax.experimental.pallas import tpu as pltpu
from jax.experimental.pallas import tpu_sc as plsc

N = 10000
D = 128
E = 320000
NUM_LAYERS = 3

NC = 2
NS = 16
NW = NC * NS
CHUNK = 64
NCH = 160
QCH = 40
EP = NW * NCH * CHUNK
NP = 10240
RPT = NP // NS


def _sc_segment_sum(m, src3, dst3, zeros):
    mesh = plsc.VectorSubcoreMesh(core_axis_name="c", subcore_axis_name="s")

    @functools.partial(
        pl.kernel,
        out_type=jax.ShapeDtypeStruct((NC * NP, D), jnp.float32),
        mesh=mesh,
        scratch_types=[
            pltpu.VMEM((NCH * CHUNK,), jnp.int32),
            pltpu.VMEM((QCH, CHUNK), jnp.int32),
            pltpu.VMEM((CHUNK, D), jnp.float32),
            pltpu.VMEM((CHUNK, D), jnp.float32),
            pltpu.VMEM((CHUNK, D), jnp.float32),
            pltpu.VMEM((CHUNK, D), jnp.float32),
            pltpu.VMEM_SHARED((NP, D), jnp.float32),
            pltpu.SemaphoreType.DMA,
            pltpu.SemaphoreType.DMA,
            pltpu.SemaphoreType.DMA,
            pltpu.SemaphoreType.DMA,
        ],
    )
    def seg(m_hbm, src_hbm, dst_hbm, z_hbm, out_hbm, src_v, dst_q, rows_a,
            rows_b, rows_c, rows_d, acc_sh, sem_a, sem_b, sem_c, sem_d):
        cid = lax.axis_index("c")
        sid = lax.axis_index("s")
        wid = sid * NC + cid
        pltpu.sync_copy(src_hbm.at[wid], src_v)
        row0 = sid * RPT
        pltpu.sync_copy(z_hbm.at[pl.ds(row0, RPT)], acc_sh.at[pl.ds(row0, RPT)])
        plsc.subcore_barrier()

        def sidx(j):
            return src_v.at[pl.ds(j * CHUNK, CHUNK)]

        bufs = ((rows_a, sem_a), (rows_b, sem_b), (rows_c, sem_c),
                (rows_d, sem_d))
        for b in range(3):
            pltpu.async_copy(m_hbm.at[sidx(b)], bufs[b][0], bufs[b][1])

        for q in range(NCH // QCH):
            pltpu.sync_copy(dst_hbm.at[wid, pl.ds(q * QCH, QCH)], dst_q)

            def body(u, carry, q=q):
                for b in range(4):
                    j = q * QCH + 4 * u + b
                    nbuf, nsem = bufs[(b + 3) % 4]

                    @pl.when(j + 3 < NCH)
                    def _():
                        pltpu.async_copy(m_hbm.at[sidx(j + 3)], nbuf, nsem)

                    cbuf, csem = bufs[b]
                    pltpu.make_async_copy(m_hbm.at[sidx(j)], cbuf, csem).wait()
                    pltpu.sync_copy(cbuf, acc_sh.at[dst_q.at[4 * u + b]],
                                    add=True)
                return carry

            lax.fori_loop(0, QCH // 4, body, 0)
        plsc.subcore_barrier()
        pltpu.sync_copy(acc_sh.at[pl.ds(row0, RPT)],
                        out_hbm.at[pl.ds(cid * NP + row0, RPT)])

    return seg(m, src3, dst3, zeros)


BLK = 1000


def _gru_proj_body(p0, p1, gh, h, wih, bih, wn, h_out, m_out):
    agg = p0[...] + p1[...]
    gi = jnp.dot(agg, wih[...], preferred_element_type=jnp.float32) + bih[...]
    ghv = gh[...]
    r = jax.nn.sigmoid(gi[:, :D] + ghv[:, :D])
    z = jax.nn.sigmoid(gi[:, D:2 * D] + ghv[:, D:2 * D])
    n = jnp.tanh(gi[:, 2 * D:] + r * ghv[:, 2 * D:])
    hn = (1.0 - z) * n + z * h[...]
    h_out[...] = hn
    m_out[...] = jnp.dot(hn, wn[...], preferred_element_type=jnp.float32)


def _tc_gru_proj(p0, p1, gh, h, wihT, bih, wnext):
    row = pl.BlockSpec((BLK, D), lambda i: (i, 0))
    row3 = pl.BlockSpec((BLK, 3 * D), lambda i: (i, 0))
    full = lambda shape: pl.BlockSpec(shape, lambda i: (0,) * len(shape))
    return pl.pallas_call(
        _gru_proj_body,
        grid=(N // BLK,),
        in_specs=[row, row, row3, row,
                  full((D, 3 * D)), full((1, 3 * D)), full((D, D))],
        out_specs=(row, row),
        out_shape=(jax.ShapeDtypeStruct((N, D), jnp.float32),
                   jax.ShapeDtypeStruct((N, D), jnp.float32)),
    )(p0, p1, gh, h, wihT, bih, wnext)


def _gh_body(h, whh, bhh, gh_out):
    gh_out[...] = jnp.dot(
        h[...], whh[...], preferred_element_type=jnp.float32) + bhh[...]


def _tc_gh(h, whhT, bhh):
    row = pl.BlockSpec((BLK, D), lambda i: (i, 0))
    full = lambda shape: pl.BlockSpec(shape, lambda i: (0,) * len(shape))
    return pl.pallas_call(
        _gh_body,
        grid=(N // BLK,),
        in_specs=[row, full((D, 3 * D)), full((1, 3 * D))],
        out_specs=pl.BlockSpec((BLK, 3 * D), lambda i: (i, 0)),
        out_shape=jax.ShapeDtypeStruct((N, 3 * D), jnp.float32),
    )(h, whhT, bhh)


def _proj_body(h, w, m_out):
    m_out[...] = jnp.dot(h[...], w[...], preferred_element_type=jnp.float32)


def _tc_proj(h, w):
    row = pl.BlockSpec((BLK, D), lambda i: (i, 0))
    return pl.pallas_call(
        _proj_body,
        grid=(N // BLK,),
        in_specs=[row, pl.BlockSpec((D, D), lambda i: (0, 0))],
        out_specs=row,
        out_shape=jax.ShapeDtypeStruct((N, D), jnp.float32),
    )(h, w)


def kernel(x, edge_index, weight, w_ih, w_hh, b_ih, b_hh):
    pad = EP - E
    pad_src = jnp.arange(pad, dtype=jnp.int32) % N
    src3 = jnp.concatenate([edge_index[0], pad_src]).reshape(NW, NCH * CHUNK)
    pad_dst = N + (jnp.arange(pad, dtype=jnp.int32) % (NP - N))
    dst3 = jnp.concatenate([edge_index[1], pad_dst]).reshape(NW, NCH, CHUNK)
    zeros = jnp.zeros((NP, D), jnp.float32)
    wihT = jnp.transpose(w_ih, (0, 2, 1))
    whhT = jnp.transpose(w_hh, (0, 2, 1))
    bih2 = b_ih.reshape(NUM_LAYERS, 1, 3 * D)
    bhh2 = b_hh.reshape(NUM_LAYERS, 1, 3 * D)

    h = x
    m = _tc_proj(h, weight[0])
    for i in range(NUM_LAYERS):
        parts = _sc_segment_sum(m, src3, dst3, zeros)
        gh = _tc_gh(h, whhT[i], bhh2[i])
        wnext = weight[i + 1] if i + 1 < NUM_LAYERS else weight[0]
        h, m = _tc_gru_proj(parts[:N], parts[NP:NP + N], gh, h, wihT[i],
                            bih2[i], wnext)
    return h


# speedup vs baseline: 1.0821x; 1.0821x over previous
import functools

import jax
import jax.numpy as jnp
from jax import lax
from jax.experimental import pallas as pl
from jax.experimental.pallas import tpu as pltpu
from jax.experimental.pallas import tpu_sc as plsc

N = 10000
D = 128
E = 320000
NUM_LAYERS = 3

NC = 2
NS = 16
NW = NC * NS
CHUNK = 64
NCH = 160
QCH = 40
EP = NW * NCH * CHUNK
NP = 10240
RPT = NP // NS


def _sc_segment_sum(m, src3, dst3, zeros):
    mesh = plsc.VectorSubcoreMesh(core_axis_name="c", subcore_axis_name="s")

    @functools.partial(
        pl.kernel,
        out_type=jax.ShapeDtypeStruct((NC, NP, D), jnp.float32),
        mesh=mesh,
        scratch_types=[
            pltpu.VMEM((NCH * CHUNK,), jnp.int32),
            pltpu.VMEM((QCH, CHUNK), jnp.int32),
            pltpu.VMEM((CHUNK, D), jnp.float32),
            pltpu.VMEM((CHUNK, D), jnp.float32),
            pltpu.VMEM((CHUNK, D), jnp.float32),
            pltpu.VMEM((CHUNK, D), jnp.float32),
            pltpu.VMEM_SHARED((NP, D), jnp.float32),
            pltpu.SemaphoreType.DMA,
            pltpu.SemaphoreType.DMA,
            pltpu.SemaphoreType.DMA,
            pltpu.SemaphoreType.DMA,
        ],
    )
    def seg(m_hbm, src_hbm, dst_hbm, z_hbm, out_hbm, src_v, dst_q, rows_a,
            rows_b, rows_c, rows_d, acc_sh, sem_a, sem_b, sem_c, sem_d):
        cid = lax.axis_index("c")
        sid = lax.axis_index("s")
        wid = sid * NC + cid
        pltpu.sync_copy(src_hbm.at[wid], src_v)
        row0 = sid * RPT
        pltpu.sync_copy(z_hbm.at[pl.ds(row0, RPT)], acc_sh.at[pl.ds(row0, RPT)])
        plsc.subcore_barrier()

        def sidx(j):
            return src_v.at[pl.ds(j * CHUNK, CHUNK)]

        bufs = ((rows_a, sem_a), (rows_b, sem_b), (rows_c, sem_c),
                (rows_d, sem_d))
        for b in range(3):
            pltpu.async_copy(m_hbm.at[sidx(b)], bufs[b][0], bufs[b][1])

        for q in range(NCH // QCH):
            pltpu.sync_copy(dst_hbm.at[wid, pl.ds(q * QCH, QCH)], dst_q)

            def body(u, carry, q=q):
                for b in range(4):
                    j = q * QCH + 4 * u + b
                    nbuf, nsem = bufs[(b + 3) % 4]

                    @pl.when(j + 3 < NCH)
                    def _():
                        pltpu.async_copy(m_hbm.at[sidx(j + 3)], nbuf, nsem)

                    cbuf, csem = bufs[b]
                    pltpu.make_async_copy(m_hbm.at[sidx(j)], cbuf, csem).wait()
                    pltpu.sync_copy(cbuf, acc_sh.at[dst_q.at[4 * u + b]],
                                    add=True)
                return carry

            lax.fori_loop(0, QCH // 4, body, 0)
        plsc.subcore_barrier()
        pltpu.sync_copy(acc_sh.at[pl.ds(row0, RPT)],
                        out_hbm.at[cid, pl.ds(row0, RPT)])

    return seg(m, src3, dst3, zeros)


BLK = 1000


def _gru_proj_body(p0, p1, h, wih, whh, bih, bhh, wn, h_out, m_out):
    agg = p0[0] + p1[0]
    gi = jnp.dot(agg, wih[...], preferred_element_type=jnp.float32) + bih[...]
    gh = jnp.dot(h[...], whh[...], preferred_element_type=jnp.float32) + bhh[...]
    r = jax.nn.sigmoid(gi[:, :D] + gh[:, :D])
    z = jax.nn.sigmoid(gi[:, D:2 * D] + gh[:, D:2 * D])
    n = jnp.tanh(gi[:, 2 * D:] + r * gh[:, 2 * D:])
    hn = (1.0 - z) * n + z * h[...]
    h_out[...] = hn
    m_out[...] = jnp.dot(hn, wn[...], preferred_element_type=jnp.float32)


def _tc_gru_proj(parts, h, wihT, whhT, bih, bhh, wnext):
    row = pl.BlockSpec((BLK, D), lambda i: (i, 0))
    part0 = pl.BlockSpec((1, BLK, D), lambda i: (0, i, 0))
    part1 = pl.BlockSpec((1, BLK, D), lambda i: (1, i, 0))
    full = lambda shape: pl.BlockSpec(shape, lambda i: (0,) * len(shape))
    return pl.pallas_call(
        _gru_proj_body,
        grid=(N // BLK,),
        in_specs=[part0, part1, row,
                  full((D, 3 * D)), full((D, 3 * D)),
                  full((1, 3 * D)), full((1, 3 * D)),
                  full((D, D))],
        out_specs=(row, row),
        out_shape=(jax.ShapeDtypeStruct((N, D), jnp.float32),
                   jax.ShapeDtypeStruct((N, D), jnp.float32)),
    )(parts, parts, h, wihT, whhT, bih, bhh, wnext)


def _proj_body(h, w, m_out):
    m_out[...] = jnp.dot(h[...], w[...], preferred_element_type=jnp.float32)


def _tc_proj(h, w):
    row = pl.BlockSpec((BLK, D), lambda i: (i, 0))
    return pl.pallas_call(
        _proj_body,
        grid=(N // BLK,),
        in_specs=[row, pl.BlockSpec((D, D), lambda i: (0, 0))],
        out_specs=row,
        out_shape=jax.ShapeDtypeStruct((N, D), jnp.float32),
    )(h, w)


def kernel(x, edge_index, weight, w_ih, w_hh, b_ih, b_hh):
    pad = EP - E
    pad_src = jnp.arange(pad, dtype=jnp.int32) % N
    src3 = jnp.concatenate([edge_index[0], pad_src]).reshape(NW, NCH * CHUNK)
    pad_dst = N + (jnp.arange(pad, dtype=jnp.int32) % (NP - N))
    dst3 = jnp.concatenate([edge_index[1], pad_dst]).reshape(NW, NCH, CHUNK)
    zeros = jnp.zeros((NP, D), jnp.float32)
    wihT = jnp.transpose(w_ih, (0, 2, 1))
    whhT = jnp.transpose(w_hh, (0, 2, 1))
    bih2 = b_ih.reshape(NUM_LAYERS, 1, 3 * D)
    bhh2 = b_hh.reshape(NUM_LAYERS, 1, 3 * D)

    h = x
    m = _tc_proj(h, weight[0])
    for i in range(NUM_LAYERS):
        parts = _sc_segment_sum(m, src3, dst3, zeros)
        wnext = weight[i + 1] if i + 1 < NUM_LAYERS else weight[0]
        h, m = _tc_gru_proj(parts, h, wihT[i], whhT[i],
                            bih2[i], bhh2[i], wnext)
    return h


# speedup vs baseline: 1.1180x; 1.0332x over previous
import functools

import jax
import jax.numpy as jnp
from jax import lax
from jax.experimental import pallas as pl
from jax.experimental.pallas import tpu as pltpu
from jax.experimental.pallas import tpu_sc as plsc

N = 10000
D = 128
E = 320000
NUM_LAYERS = 3

NC = 2
NS = 16
NW = NC * NS
CHUNK = 64
NCH = 160
QCH = 40
EP = NW * NCH * CHUNK
NP = 10240
RPT = NP // NS


def _sc_segment_sum(m, src3, dst3, zeros):
    mesh = plsc.VectorSubcoreMesh(core_axis_name="c", subcore_axis_name="s")

    @functools.partial(
        pl.kernel,
        out_type=jax.ShapeDtypeStruct((NC, NP, D), jnp.float32),
        mesh=mesh,
        scratch_types=[
            pltpu.VMEM((NCH * CHUNK,), jnp.int32),
            pltpu.VMEM((QCH, CHUNK), jnp.int32),
            pltpu.VMEM((CHUNK, D), jnp.float32),
            pltpu.VMEM((CHUNK, D), jnp.float32),
            pltpu.VMEM((CHUNK, D), jnp.float32),
            pltpu.VMEM((CHUNK, D), jnp.float32),
            pltpu.VMEM_SHARED((NP, D), jnp.float32),
            pltpu.SemaphoreType.DMA,
            pltpu.SemaphoreType.DMA,
            pltpu.SemaphoreType.DMA,
            pltpu.SemaphoreType.DMA,
        ],
    )
    def seg(m_hbm, src_hbm, dst_hbm, z_hbm, out_hbm, src_v, dst_q, rows_a,
            rows_b, rows_c, rows_d, acc_sh, sem_a, sem_b, sem_c, sem_d):
        cid = lax.axis_index("c")
        sid = lax.axis_index("s")
        wid = sid * NC + cid
        pltpu.sync_copy(src_hbm.at[wid], src_v)
        row0 = sid * RPT
        pltpu.sync_copy(z_hbm.at[pl.ds(row0, RPT)], acc_sh.at[pl.ds(row0, RPT)])
        plsc.subcore_barrier()

        def sidx(j):
            return src_v.at[pl.ds(j * CHUNK, CHUNK)]

        bufs = ((rows_a, sem_a), (rows_b, sem_b), (rows_c, sem_c),
                (rows_d, sem_d))
        for b in range(3):
            pltpu.async_copy(m_hbm.at[sidx(b)], bufs[b][0], bufs[b][1])

        for q in range(NCH // QCH):
            pltpu.sync_copy(dst_hbm.at[wid, pl.ds(q * QCH, QCH)], dst_q)

            def body(u, carry, q=q):
                for b in range(4):
                    j = q * QCH + 4 * u + b
                    nbuf, nsem = bufs[(b + 3) % 4]

                    @pl.when(j + 3 < NCH)
                    def _():
                        pltpu.async_copy(m_hbm.at[sidx(j + 3)], nbuf, nsem)

                    cbuf, csem = bufs[b]
                    pltpu.make_async_copy(m_hbm.at[sidx(j)], cbuf, csem).wait()
                    pltpu.sync_copy(cbuf, acc_sh.at[dst_q.at[4 * u + b]],
                                    add=True)
                return carry

            lax.fori_loop(0, QCH // 4, body, 0)
        plsc.subcore_barrier()
        pltpu.sync_copy(acc_sh.at[pl.ds(row0, RPT)],
                        out_hbm.at[cid, pl.ds(row0, RPT)])

    return seg(m, src3, dst3, zeros)


BLK = 2000


def _gru_core(p0, p1, h, wih, whh, bih, bhh):
    agg = p0[0] + p1[0]
    gi = jnp.dot(agg, wih[...], preferred_element_type=jnp.float32) + bih[...]
    gh = jnp.dot(h[...], whh[...], preferred_element_type=jnp.float32) + bhh[...]
    r = jax.nn.sigmoid(gi[:, :D] + gh[:, :D])
    z = jax.nn.sigmoid(gi[:, D:2 * D] + gh[:, D:2 * D])
    n = jnp.tanh(gi[:, 2 * D:] + r * gh[:, 2 * D:])
    return (1.0 - z) * n + z * h[...]


def _gru_proj_body(p0, p1, h, wih, whh, bih, bhh, wn, h_out, m_out):
    hn = _gru_core(p0, p1, h, wih, whh, bih, bhh)
    h_out[...] = hn
    m_out[...] = jnp.dot(hn, wn[...], preferred_element_type=jnp.float32)


def _gru_last_body(p0, p1, h, wih, whh, bih, bhh, h_out):
    h_out[...] = _gru_core(p0, p1, h, wih, whh, bih, bhh)


def _tc_gru(parts, h, wihT, whhT, bih, bhh, wnext=None):
    row = pl.BlockSpec((BLK, D), lambda i: (i, 0))
    part0 = pl.BlockSpec((1, BLK, D), lambda i: (0, i, 0))
    part1 = pl.BlockSpec((1, BLK, D), lambda i: (1, i, 0))
    full = lambda shape: pl.BlockSpec(shape, lambda i: (0,) * len(shape))
    specs = [part0, part1, row,
             full((D, 3 * D)), full((D, 3 * D)),
             full((1, 3 * D)), full((1, 3 * D))]
    args = [parts, parts, h, wihT, whhT, bih, bhh]
    if wnext is None:
        return pl.pallas_call(
            _gru_last_body,
            grid=(N // BLK,),
            in_specs=specs,
            out_specs=row,
            out_shape=jax.ShapeDtypeStruct((N, D), jnp.float32),
        )(*args)
    return pl.pallas_call(
        _gru_proj_body,
        grid=(N // BLK,),
        in_specs=specs + [full((D, D))],
        out_specs=(row, row),
        out_shape=(jax.ShapeDtypeStruct((N, D), jnp.float32),
                   jax.ShapeDtypeStruct((N, D), jnp.float32)),
    )(*args, wnext)


def _proj_body(h, w, m_out):
    m_out[...] = jnp.dot(h[...], w[...], preferred_element_type=jnp.float32)


def _tc_proj(h, w):
    row = pl.BlockSpec((BLK, D), lambda i: (i, 0))
    return pl.pallas_call(
        _proj_body,
        grid=(N // BLK,),
        in_specs=[row, pl.BlockSpec((D, D), lambda i: (0, 0))],
        out_specs=row,
        out_shape=jax.ShapeDtypeStruct((N, D), jnp.float32),
    )(h, w)


def kernel(x, edge_index, weight, w_ih, w_hh, b_ih, b_hh):
    pad = EP - E
    pad_src = jnp.arange(pad, dtype=jnp.int32) % N
    src3 = jnp.concatenate([edge_index[0], pad_src]).reshape(NW, NCH * CHUNK)
    pad_dst = N + (jnp.arange(pad, dtype=jnp.int32) % (NP - N))
    dst3 = jnp.concatenate([edge_index[1], pad_dst]).reshape(NW, NCH, CHUNK)
    zeros = jnp.zeros((NP, D), jnp.float32)
    wihT = jnp.transpose(w_ih, (0, 2, 1))
    whhT = jnp.transpose(w_hh, (0, 2, 1))
    bih2 = b_ih.reshape(NUM_LAYERS, 1, 3 * D)
    bhh2 = b_hh.reshape(NUM_LAYERS, 1, 3 * D)

    h = x
    m = _tc_proj(h, weight[0])
    for i in range(NUM_LAYERS):
        parts = _sc_segment_sum(m, src3, dst3, zeros)
        if i + 1 < NUM_LAYERS:
            h, m = _tc_gru(parts, h, wihT[i], whhT[i], bih2[i], bhh2[i],
                           weight[i + 1])
        else:
            h = _tc_gru(parts, h, wihT[i], whhT[i], bih2[i], bhh2[i])
    return h


# speedup vs baseline: 1.1252x; 1.0064x over previous
import functools

import jax
import jax.numpy as jnp
from jax import lax
from jax.experimental import pallas as pl
from jax.experimental.pallas import tpu as pltpu
from jax.experimental.pallas import tpu_sc as plsc

N = 10000
D = 128
E = 320000
NUM_LAYERS = 3

NC = 2
NS = 16
NW = NC * NS
CHUNK = 64
NCH = 160
QCH = 40
EP = NW * NCH * CHUNK
NP = 10240
RPT = NP // NS


def _sc_segment_sum(m, src3, dst3, zeros):
    mesh = plsc.VectorSubcoreMesh(core_axis_name="c", subcore_axis_name="s")

    @functools.partial(
        pl.kernel,
        out_type=jax.ShapeDtypeStruct((NC, NP, D), jnp.float32),
        mesh=mesh,
        scratch_types=[
            pltpu.VMEM((NCH * CHUNK,), jnp.int32),
            pltpu.VMEM((QCH, CHUNK), jnp.int32),
            pltpu.VMEM((CHUNK, D), jnp.float32),
            pltpu.VMEM((CHUNK, D), jnp.float32),
            pltpu.VMEM((CHUNK, D), jnp.float32),
            pltpu.VMEM((CHUNK, D), jnp.float32),
            pltpu.VMEM_SHARED((NP, D), jnp.float32),
            pltpu.SemaphoreType.DMA,
            pltpu.SemaphoreType.DMA,
            pltpu.SemaphoreType.DMA,
            pltpu.SemaphoreType.DMA,
        ],
    )
    def seg(m_hbm, src_hbm, dst_hbm, z_hbm, out_hbm, src_v, dst_q, rows_a,
            rows_b, rows_c, rows_d, acc_sh, sem_a, sem_b, sem_c, sem_d):
        cid = lax.axis_index("c")
        sid = lax.axis_index("s")
        wid = sid * NC + cid
        pltpu.sync_copy(src_hbm.at[wid], src_v)

        def sidx(j):
            return src_v.at[pl.ds(j * CHUNK, CHUNK)]

        bufs = ((rows_a, sem_a), (rows_b, sem_b), (rows_c, sem_c),
                (rows_d, sem_d))
        for b in range(3):
            pltpu.async_copy(m_hbm.at[sidx(b)], bufs[b][0], bufs[b][1])

        row0 = sid * RPT
        pltpu.sync_copy(z_hbm.at[pl.ds(row0, RPT)], acc_sh.at[pl.ds(row0, RPT)])
        plsc.subcore_barrier()

        for q in range(NCH // QCH):
            pltpu.sync_copy(dst_hbm.at[wid, pl.ds(q * QCH, QCH)], dst_q)

            def body(u, carry, q=q):
                for b in range(4):
                    j = q * QCH + 4 * u + b
                    nbuf, nsem = bufs[(b + 3) % 4]

                    @pl.when(j + 3 < NCH)
                    def _():
                        pltpu.async_copy(m_hbm.at[sidx(j + 3)], nbuf, nsem)

                    cbuf, csem = bufs[b]
                    pltpu.make_async_copy(m_hbm.at[sidx(j)], cbuf, csem).wait()
                    pltpu.sync_copy(cbuf, acc_sh.at[dst_q.at[4 * u + b]],
                                    add=True)
                return carry

            lax.fori_loop(0, QCH // 4, body, 0)
        plsc.subcore_barrier()
        pltpu.sync_copy(acc_sh.at[pl.ds(row0, RPT)],
                        out_hbm.at[cid, pl.ds(row0, RPT)])

    return seg(m, src3, dst3, zeros)


BLK = 2000


def _gru_core(p0, p1, h, wih, whh, bih, bhh):
    agg = p0[0] + p1[0]
    gi = jnp.dot(agg, wih[...], preferred_element_type=jnp.float32) + bih[...]
    gh = jnp.dot(h[...], whh[...], preferred_element_type=jnp.float32) + bhh[...]
    r = jax.nn.sigmoid(gi[:, :D] + gh[:, :D])
    z = jax.nn.sigmoid(gi[:, D:2 * D] + gh[:, D:2 * D])
    n = jnp.tanh(gi[:, 2 * D:] + r * gh[:, 2 * D:])
    return (1.0 - z) * n + z * h[...]


def _gru_proj_body(p0, p1, h, wih, whh, bih, bhh, wn, h_out, m_out):
    hn = _gru_core(p0, p1, h, wih, whh, bih, bhh)
    h_out[...] = hn
    m_out[...] = jnp.dot(hn, wn[...], preferred_element_type=jnp.float32)


def _gru_last_body(p0, p1, h, wih, whh, bih, bhh, h_out):
    h_out[...] = _gru_core(p0, p1, h, wih, whh, bih, bhh)


def _tc_gru(parts, h, wihT, whhT, bih, bhh, wnext=None):
    row = pl.BlockSpec((BLK, D), lambda i: (i, 0))
    part0 = pl.BlockSpec((1, BLK, D), lambda i: (0, i, 0))
    part1 = pl.BlockSpec((1, BLK, D), lambda i: (1, i, 0))
    full = lambda shape: pl.BlockSpec(shape, lambda i: (0,) * len(shape))
    specs = [part0, part1, row,
             full((D, 3 * D)), full((D, 3 * D)),
             full((1, 3 * D)), full((1, 3 * D))]
    args = [parts, parts, h, wihT, whhT, bih, bhh]
    if wnext is None:
        return pl.pallas_call(
            _gru_last_body,
            grid=(N // BLK,),
            in_specs=specs,
            out_specs=row,
            out_shape=jax.ShapeDtypeStruct((N, D), jnp.float32),
        )(*args)
    return pl.pallas_call(
        _gru_proj_body,
        grid=(N // BLK,),
        in_specs=specs + [full((D, D))],
        out_specs=(row, row),
        out_shape=(jax.ShapeDtypeStruct((N, D), jnp.float32),
                   jax.ShapeDtypeStruct((N, D), jnp.float32)),
    )(*args, wnext)


def _proj_body(h, w, m_out):
    m_out[...] = jnp.dot(h[...], w[...], preferred_element_type=jnp.float32)


def _tc_proj(h, w):
    row = pl.BlockSpec((BLK, D), lambda i: (i, 0))
    return pl.pallas_call(
        _proj_body,
        grid=(N // BLK,),
        in_specs=[row, pl.BlockSpec((D, D), lambda i: (0, 0))],
        out_specs=row,
        out_shape=jax.ShapeDtypeStruct((N, D), jnp.float32),
    )(h, w)


def kernel(x, edge_index, weight, w_ih, w_hh, b_ih, b_hh):
    pad = EP - E
    pad_src = jnp.arange(pad, dtype=jnp.int32) % N
    src3 = jnp.concatenate([edge_index[0], pad_src]).reshape(NW, NCH * CHUNK)
    pad_dst = N + (jnp.arange(pad, dtype=jnp.int32) % (NP - N))
    dst3 = jnp.concatenate([edge_index[1], pad_dst]).reshape(NW, NCH, CHUNK)
    zeros = jnp.zeros((NP, D), jnp.float32)
    wihT = jnp.transpose(w_ih, (0, 2, 1))
    whhT = jnp.transpose(w_hh, (0, 2, 1))
    bih2 = b_ih.reshape(NUM_LAYERS, 1, 3 * D)
    bhh2 = b_hh.reshape(NUM_LAYERS, 1, 3 * D)

    h = x
    m = _tc_proj(h, weight[0])
    for i in range(NUM_LAYERS):
        parts = _sc_segment_sum(m, src3, dst3, zeros)
        if i + 1 < NUM_LAYERS:
            h, m = _tc_gru(parts, h, wihT[i], whhT[i], bih2[i], bhh2[i],
                           weight[i + 1])
        else:
            h = _tc_gru(parts, h, wihT[i], whhT[i], bih2[i], bhh2[i])
    return h
